# baseline (speedup 1.0000x reference)
import functools
import jax
import jax.numpy as jnp
from jax import lax
from jax.experimental import pallas as pl
from jax.experimental.pallas import tpu as pltpu
from jax.experimental.pallas import tpu_sc as plsc

H = 128
ALPHA = 0.2
K_APPNP = 10
N_T = 10000
N_C = 50000


def _proj_table_body(x_ref, wcat_ref, qr_ref, out_ref, w_ref):
    x = x_ref[...]
    out_ref[...] = jnp.dot(x, wcat_ref[...], preferred_element_type=jnp.float32)
    s = jnp.dot(x, qr_ref[...], preferred_element_type=jnp.float32)
    xn = jnp.sqrt(jnp.sum(x * x, axis=1, keepdims=True))
    w_ref[...] = jnp.maximum(s, 0.0) / jnp.maximum(xn, 1e-12)


def _proj_table(x_table, wcat, qr, blk=2000):
    nb = N_T // blk
    return pl.pallas_call(
        _proj_table_body,
        grid=(nb,),
        in_specs=[
            pl.BlockSpec((blk, 512), lambda i: (i, 0)),
            pl.BlockSpec((512, wcat.shape[1]), lambda i: (0, 0)),
            pl.BlockSpec((512, 1), lambda i: (0, 0)),
        ],
        out_specs=[
            pl.BlockSpec((blk, wcat.shape[1]), lambda i: (i, 0)),
            pl.BlockSpec((blk, 1), lambda i: (i, 0)),
        ],
        out_shape=[
            jax.ShapeDtypeStruct((N_T, wcat.shape[1]), jnp.float32),
            jax.ShapeDtypeStruct((N_T, 1), jnp.float32),
        ],
    )(x_table, wcat, qr)


def _proj_col_body(x_ref, wcat_ref, out_ref):
    out_ref[...] = jnp.dot(x_ref[...], wcat_ref[...],
                           preferred_element_type=jnp.float32)


def _proj_col(x_col, wcat, blk=2000):
    nb = N_C // blk
    return pl.pallas_call(
        _proj_col_body,
        grid=(nb,),
        in_specs=[
            pl.BlockSpec((blk, H), lambda i: (i, 0)),
            pl.BlockSpec((H, wcat.shape[1]), lambda i: (0, 0)),
        ],
        out_specs=pl.BlockSpec((blk, wcat.shape[1]), lambda i: (i, 0)),
        out_shape=jax.ShapeDtypeStruct((N_C, wcat.shape[1]), jnp.float32),
    )(x_col, wcat)


def _post_col_body(s_ref, m_ref, v_ref, a_ref):
    xc1 = jnp.maximum(s_ref[...] + m_ref[...], 0.0)
    a_ref[...] = jnp.dot(xc1, v_ref[...], preferred_element_type=jnp.float32)


def _post_col(S_c, Mc, v_rhc, blk=2000):
    nb = N_C // blk
    return pl.pallas_call(
        _post_col_body,
        grid=(nb,),
        in_specs=[
            pl.BlockSpec((blk, H), lambda i: (i, 0)),
            pl.BlockSpec((blk, H), lambda i: (i, 0)),
            pl.BlockSpec((H, 1), lambda i: (0, 0)),
        ],
        out_specs=pl.BlockSpec((blk, 1), lambda i: (i, 0)),
        out_shape=jax.ShapeDtypeStruct((N_C, 1), jnp.float32),
    )(S_c, Mc, v_rhc)


def _post_table_body(s_ref, m_ref, w_ref, cnt_ref, vv_ref, consts_ref,
                     a_ref, t_ref, d_ref):
    xt1 = jnp.maximum(s_ref[...] + m_ref[...], 0.0)
    av = jnp.dot(xt1, vv_ref[...], preferred_element_type=jnp.float32)
    a_ref[...] = av[:, 0:1]
    bias2 = consts_ref[0, 0]
    qvu = consts_ref[0, 1]
    t_ref[...] = av[:, 1:2] + bias2 + qvu * w_ref[...]
    d_ref[...] = lax.rsqrt(cnt_ref[...] + 1.0)


def _post_table(S_t, Mt, w, cnt, vv, consts, blk=2000):
    nb = N_T // blk
    return pl.pallas_call(
        _post_table_body,
        grid=(nb,),
        in_specs=[
            pl.BlockSpec((blk, H), lambda i: (i, 0)),
            pl.BlockSpec((blk, H), lambda i: (i, 0)),
            pl.BlockSpec((blk, 1), lambda i: (i, 0)),
            pl.BlockSpec((blk, 1), lambda i: (i, 0)),
            pl.BlockSpec((H, 2), lambda i: (0, 0)),
            pl.BlockSpec((1, 2), lambda i: (0, 0)),
        ],
        out_specs=[
            pl.BlockSpec((blk, 1), lambda i: (i, 0)),
            pl.BlockSpec((blk, 1), lambda i: (i, 0)),
            pl.BlockSpec((blk, 1), lambda i: (i, 0)),
        ],
        out_shape=[
            jax.ShapeDtypeStruct((N_T, 1), jnp.float32),
            jax.ShapeDtypeStruct((N_T, 1), jnp.float32),
            jax.ShapeDtypeStruct((N_T, 1), jnp.float32),
        ],
    )(S_t, Mt, w, cnt, vv, consts)


RNG = 1920
ACC = 2048
LDUMP = 2000
C_DUMP = 50100
N_CR = 28
N_TR = 6
P_C_OUT = N_CR * RNG
P_T_OUT = N_TR * RNG
E_HC_P = 51200
E_CC_P = 512000
STG = 800


def _sc_segmeans(tabs, edges):
    mesh = plsc.VectorSubcoreMesh(core_axis_name="c", subcore_axis_name="s")

    c_cfg = [('hc', E_HC_P), ('cs', E_CC_P), ('ns', E_CC_P), ('ds', E_CC_P)]
    t_cfg = [('rhc', E_HC_P), ('ts', E_TS)]

    @functools.partial(
        pl.kernel, mesh=mesh,
        compiler_params=pltpu.CompilerParams(needs_layout_passes=False),
        out_type=[
            jax.ShapeDtypeStruct((P_C_OUT, H), jnp.float32),
            jax.ShapeDtypeStruct((P_T_OUT, H), jnp.float32),
            jax.ShapeDtypeStruct((P_T_OUT,), jnp.float32),
            jax.ShapeDtypeStruct((P_T_OUT,), jnp.float32),
            jax.ShapeDtypeStruct((P_T_OUT,), jnp.float32),
        ],
        scratch_types=[
            pltpu.VMEM((STG,), jnp.int32),
            pltpu.VMEM((STG,), jnp.int32),
            pltpu.VMEM((256,), jnp.int32),
            pltpu.VMEM((256,), jnp.int32),
            pltpu.VMEM((128,), jnp.int32),
            pltpu.VMEM((128,), jnp.int32),
            pltpu.VMEM((128, H), jnp.float32),
            pltpu.VMEM((4 * ACC,), jnp.float32),
            pltpu.VMEM((128, H), jnp.float32),
            pltpu.VMEM((40, H), jnp.float32),
            pltpu.VMEM((1024,), jnp.float32),
            pltpu.VMEM((128,), jnp.float32),
            pltpu.VMEM_SHARED((4, ACC, H), jnp.float32),
            pltpu.VMEM_SHARED((N_SUB, 4 * ACC), jnp.float32),
            pltpu.SemaphoreType.DMA,
        ],
    )
    def k(hc_t, cs_t, ns_t, ds_t, rhc_t, ts_t,
          hc_s, hc_d, cs_s, cs_d, ns_s, ns_d, ds_s, ds_d,
          rhc_s, rhc_d, ts_s, ts_d,
          sc_out, st_out, invr_out, invt_out, cntts_out,
          stg_s, stg_d, gq, sq, gi, si, rv, cacc, rv40, outb, invb, ctmp,
          vacc, cslots, sem):
        cid = lax.axis_index("c")
        sid = lax.axis_index("s")

        def zero_vec(ref, n):
            def zb(i, _):
                ref[pl.ds(i * 16, 16)] = jnp.zeros((16,), jnp.float32)
                return 0
            lax.fori_loop(0, n // 16, zb, 0)

        def flush(tab_hbm, t):
            for j in range(8):
                s = pl.ds(j * 16, 16)
                gi[s] = gq[s]
                si[s] = sq[s]
            pltpu.async_copy(tab_hbm.at[gi], rv, sem).wait()
            pltpu.sync_copy(rv, vacc.at[t].at[si], add=True)

        def scan_type(t, tab_hbm, src_hbm, dst_hbm, e_pad, lo):
            ch = e_pad // N_SUB
            ebase = sid * ch
            n_stg = ch // STG

            def stage_loop(st, nf):
                pltpu.sync_copy(src_hbm.at[pl.ds(ebase + st * STG, STG)], stg_s)
                pltpu.sync_copy(dst_hbm.at[pl.ds(ebase + st * STG, STG)], stg_d)

                def gb(g, nf):
                    s = pl.ds(g * 16, 16)
                    sr = stg_s[s]
                    d = stg_d[s]
                    dl = d - lo
                    m = jnp.logical_and(dl >= 0, dl < RNG)
                    ci = jnp.where(m, dl, LDUMP) + t * ACC
                    plsc.addupdate_scatter(cacc, [ci], jnp.ones((16,), jnp.float32))
                    plsc.store_compressed(gq.at[pl.ds(nf, 16)], sr, mask=m)
                    plsc.store_compressed(sq.at[pl.ds(nf, 16)],
                                          jnp.where(m, dl, LDUMP), mask=m)
                    nf2 = nf + jnp.sum(m.astype(jnp.int32))

                    @pl.when(nf2 >= 113)
                    def _():
                        gq[pl.ds(nf2, 16)] = jnp.zeros((16,), jnp.int32)
                        sq[pl.ds(nf2, 16)] = jnp.full((16,), LDUMP, jnp.int32)
                        flush(tab_hbm, t)
                    return jnp.where(nf2 >= 113, 0, nf2)

                return lax.fori_loop(0, STG // 16, gb, nf)

            nf = lax.fori_loop(0, n_stg, stage_loop, jnp.int32(0))
            def pb(j, _):
                gq[pl.ds(nf + j * 16, 16)] = jnp.zeros((16,), jnp.int32)
                sq[pl.ds(nf + j * 16, 16)] = jnp.full((16,), LDUMP, jnp.int32)
                return 0
            lax.fori_loop(0, 8, pb, 0)
            flush(tab_hbm, t)

        def drain(n_types, lo, souts, inv_outs, cnt_outs):
            for rep in range(2):
                tile = sid + rep * N_SUB

                @pl.when(tile < RNG // 128)
                def _():
                    r0 = tile * 128
                    for t in range(n_types):
                        def zc(i, _):
                            ctmp[pl.ds(i * 16, 16)] = jnp.zeros((16,), jnp.float32)
                            return 0
                        lax.fori_loop(0, 128 // 16, zc, 0)
                        for w in range(N_SUB):
                            pltpu.sync_copy(
                                cslots.at[w, pl.ds(t * ACC + r0, 128)],
                                invb.at[pl.ds(512, 128)])
                            def ac(i, _):
                                sl = pl.ds(i * 16, 16)
                                ctmp[sl] = ctmp[sl] + invb[pl.ds(512 + i * 16, 16)]
                                return 0
                            lax.fori_loop(0, 128 // 16, ac, 0)
                        if cnt_outs is not None and t == cnt_outs[0]:
                            pltpu.sync_copy(ctmp, cnt_outs[1].at[pl.ds(lo + r0, 128)])
                        def iv(i, _):
                            sl = pl.ds(i * 16, 16)
                            invb[pl.ds(t * 128 + i * 16, 16)] = \
                                1.0 / jnp.maximum(ctmp[sl], 1.0)
                            return 0
                        lax.fori_loop(0, 128 // 16, iv, 0)
                        if inv_outs is not None and t in inv_outs:
                            pltpu.sync_copy(invb.at[pl.ds(t * 128, 128)],
                                            inv_outs[t].at[pl.ds(lo + r0, 128)])
                    def zo(i, _):
                        rv[i // 8, pl.ds((i % 8) * 16, 16)] = \
                            jnp.zeros((16,), jnp.float32)
                        return 0
                    lax.fori_loop(0, 128 * 8, zo, 0)
                    for t in range(n_types):
                        pltpu.sync_copy(vacc.at[t, pl.ds(r0, 128)], rv40)
                        def rowb(i, _):
                            cv = invb[pl.ds(t * 128 + i, 16)]
                            c = cv[0]
                            for j in range(8):
                                sl = pl.ds(j * 16, 16)
                                rv[i, sl] = rv[i, sl] + rv40[i, sl] * c
                            return 0
                        lax.fori_loop(0, 128, rowb, 0)
                    pltpu.sync_copy(rv, souts.at[pl.ds(lo + r0, 128)])

        def zero_shared():
            def zo(i, _):
                outb[i // 8, pl.ds((i % 8) * 16, 16)] = jnp.zeros((16,), jnp.float32)
                return 0
            lax.fori_loop(0, 40 * 8, zo, 0)
            for t in range(4):
                for rep in range(2):
                    tile = sid + rep * N_SUB

                    @pl.when(tile < ACC // 128)
                    def _():
                        def zs(i, _):
                            pltpu.sync_copy(
                                outb.at[pl.ds(0, 32)],
                                vacc.at[t].at[pl.ds(tile * 128 + i * 32, 32)])
                            return 0
                        lax.fori_loop(0, 4, zs, 0)
            zero_vec(cacc, 4 * ACC)
            pltpu.sync_copy(cacc, cslots.at[sid])

        def c_round(rnd, _):
            lo = (2 * rnd + cid) * RNG
            zero_shared()
            plsc.subcore_barrier()
            scan_type(0, hc_t, hc_s, hc_d, E_HC_P, lo)
            scan_type(1, cs_t, cs_s, cs_d, E_CC_P, lo)
            scan_type(2, ns_t, ns_s, ns_d, E_CC_P, lo)
            scan_type(3, ds_t, ds_s, ds_d, E_CC_P, lo)
            pltpu.sync_copy(cacc, cslots.at[sid])
            plsc.subcore_barrier()
            drain(4, lo, sc_out, None, None)
            plsc.subcore_barrier()
            return 0
        lax.fori_loop(0, N_CR // 2, c_round, 0)

        def t_round(rnd, _):
            lo = (2 * rnd + cid) * RNG
            zero_shared()
            plsc.subcore_barrier()
            scan_type(0, rhc_t, rhc_s, rhc_d, E_HC_P, lo)
            scan_type(1, ts_t, ts_s, ts_d, E_TS, lo)
            pltpu.sync_copy(cacc, cslots.at[sid])
            plsc.subcore_barrier()
            drain(2, lo, st_out, {0: invr_out, 1: invt_out},
                  (1, cntts_out))
            plsc.subcore_barrier()
            return 0
        lax.fori_loop(0, N_TR // 2, t_round, 0)

    return k(tabs['hc'], tabs['cs'], tabs['ns'], tabs['ds'],
             tabs['rhc'], tabs['ts'],
             edges['hc'][0], edges['hc'][1], edges['cs'][0], edges['cs'][1],
             edges['ns'][0], edges['ns'][1], edges['ds'][0], edges['ds'][1],
             edges['rhc'][0], edges['rhc'][1], edges['ts'][0], edges['ts'][1])


P_T = 10240
T_DUMP = 10100
E_TS = 320000
E_RHC_P = 51200
TS_CH = E_TS // 16
RHC_CH = E_RHC_P // 16
STRIPE = P_T // 16
N_SUB = 16


def _sc_appnp(a_c, a_t_p, hb_p, dis_p, invr_p, invt_p, tsr, tsc, rr_p, rc_p):
    mesh = plsc.VectorSubcoreMesh(core_axis_name="c", subcore_axis_name="s")

    @functools.partial(
        pl.kernel, mesh=mesh,
        compiler_params=pltpu.CompilerParams(needs_layout_passes=False),
        out_type=jax.ShapeDtypeStruct((P_T,), jnp.float32),
        scratch_types=[
            pltpu.VMEM((TS_CH,), jnp.int32),
            pltpu.VMEM((TS_CH,), jnp.int32),
            pltpu.VMEM((TS_CH,), jnp.float32),
            pltpu.VMEM((RHC_CH,), jnp.int32),
            pltpu.VMEM((RHC_CH,), jnp.int32),
            pltpu.VMEM((RHC_CH,), jnp.float32),
            pltpu.VMEM((P_T,), jnp.float32),
            pltpu.VMEM((P_T,), jnp.float32),
            pltpu.VMEM((P_T,), jnp.float32),
            pltpu.VMEM((P_T,), jnp.float32),
            pltpu.VMEM((STRIPE,), jnp.float32),
            pltpu.VMEM((STRIPE,), jnp.float32),
            pltpu.VMEM((STRIPE,), jnp.float32),
            pltpu.VMEM((STRIPE,), jnp.float32),
            pltpu.VMEM_SHARED((N_SUB, P_T), jnp.float32),
            pltpu.VMEM_SHARED((P_T,), jnp.float32),
            pltpu.SemaphoreType.DMA,
        ],
    )
    def k(ac_hbm, at_hbm, hb_hbm, dis_hbm, invr_hbm, invt_hbm,
          tsr_hbm, tsc_hbm, rr_hbm, rc_hbm, out_hbm,
          er, ec, en, rr, rc, gat, at, dis, val, hacc,
          hs, ds2, tmp, tmp2, slots, sv, sem):
        cid = lax.axis_index("c")
        sid = lax.axis_index("s")

        @pl.when(cid == 0)
        def _():
            so = sid * STRIPE

            def zero_hacc():
                def zb(i, _):
                    hacc[pl.ds(i * 16, 16)] = jnp.zeros((16,), jnp.float32)
                    return 0
                lax.fori_loop(0, P_T // 16, zb, 0)

            def stripe_reduce_into(dst_scale_ref, scale_from_tmp2):
                def zb(j, _):
                    tmp[pl.ds(j * 16, 16)] = jnp.zeros((16,), jnp.float32)
                    return 0
                lax.fori_loop(0, STRIPE // 16, zb, 0)
                for t in range(N_SUB):
                    pltpu.sync_copy(slots.at[t, pl.ds(so, STRIPE)], tmp2)
                    def ab(j, _):
                        s = pl.ds(j * 16, 16)
                        tmp[s] = tmp[s] + tmp2[s]
                        return 0
                    lax.fori_loop(0, STRIPE // 16, ab, 0)

            pltpu.sync_copy(tsr_hbm.at[pl.ds(sid * TS_CH, TS_CH)], er)
            pltpu.sync_copy(tsc_hbm.at[pl.ds(sid * TS_CH, TS_CH)], ec)
            pltpu.sync_copy(rr_hbm.at[pl.ds(sid * RHC_CH, RHC_CH)], rr)
            pltpu.sync_copy(rc_hbm.at[pl.ds(sid * RHC_CH, RHC_CH)], rc)
            pltpu.sync_copy(dis_hbm, dis)
            pltpu.sync_copy(at_hbm, at)

            def nb(g, _):
                s = pl.ds(g * 16, 16)
                en[s] = (plsc.load_gather(dis, [er[s]])
                         * plsc.load_gather(dis, [ec[s]]))
                return 0
            lax.fori_loop(0, TS_CH // 16, nb, 0)
            for j in range(STRIPE // 16):
                d = dis[pl.ds(so + j * 16, 16)]
                ds2[pl.ds(j * 16, 16)] = d * d

            zero_hacc()
            cps = [pltpu.async_copy(
                ac_hbm.at[rr.at[pl.ds(kk * 128, 128)]],
                gat.at[pl.ds(kk * 128, 128)], sem)
                for kk in range(RHC_CH // 128)]
            for cp in cps:
                cp.wait()

            def rb(g, _):
                s = pl.ds(g * 16, 16)
                plsc.addupdate_scatter(hacc, [rc[s]], gat[s])
                return 0
            lax.fori_loop(0, RHC_CH // 16, rb, 0)
            pltpu.sync_copy(hacc, slots.at[sid])
            plsc.subcore_barrier()
            stripe_reduce_into(None, None)
            pltpu.sync_copy(invr_hbm.at[pl.ds(so, STRIPE)], tmp2)
            for j in range(STRIPE // 16):
                s = pl.ds(j * 16, 16)
                hs[s] = tmp[s] * tmp2[s]
            plsc.subcore_barrier()

            zero_hacc()
            def tb(g, _):
                s = pl.ds(g * 16, 16)
                plsc.addupdate_scatter(hacc, [ec[s]],
                                       plsc.load_gather(at, [er[s]]))
                return 0
            lax.fori_loop(0, TS_CH // 16, tb, 0)
            pltpu.sync_copy(hacc, slots.at[sid])
            plsc.subcore_barrier()
            stripe_reduce_into(None, None)
            pltpu.sync_copy(invt_hbm.at[pl.ds(so, STRIPE)], tmp2)
            for j in range(STRIPE // 16):
                s = pl.ds(j * 16, 16)
                hs[s] = hs[s] + tmp[s] * tmp2[s]
            pltpu.sync_copy(hb_hbm.at[pl.ds(so, STRIPE)], tmp2)
            for j in range(STRIPE // 16):
                s = pl.ds(j * 16, 16)
                hs[s] = hs[s] + tmp2[s]

            pltpu.sync_copy(hs, sv.at[pl.ds(so, STRIPE)])
            plsc.subcore_barrier()
            pltpu.sync_copy(sv, val)

            for _ in range(K_APPNP):
                zero_hacc()
                def ib(g, _):
                    s = pl.ds(g * 16, 16)
                    x = plsc.load_gather(val, [er[s]]) * en[s]
                    plsc.addupdate_scatter(hacc, [ec[s]], x)
                    return 0
                lax.fori_loop(0, TS_CH // 16, ib, 0)
                pltpu.sync_copy(hacc, slots.at[sid])
                plsc.subcore_barrier()
                stripe_reduce_into(None, None)
                for j in range(STRIPE // 16):
                    s = pl.ds(j * 16, 16)
                    sj = pl.ds(so + j * 16, 16)
                    vnew = (1.0 - ALPHA) * (tmp[s] + ds2[s] * val[sj]) \
                        + ALPHA * hs[s]
                    tmp2[s] = vnew
                pltpu.sync_copy(tmp2, sv.at[pl.ds(so, STRIPE)])
                plsc.subcore_barrier()
                pltpu.sync_copy(sv, val)

            @pl.when(sid == 0)
            def _():
                pltpu.sync_copy(val, out_hbm)

    return k(a_c, a_t_p, hb_p, dis_p, invr_p, invt_p, tsr, tsc, rr_p, rc_p)


def _counts(dst, n_dst):
    return jax.ops.segment_sum(jnp.ones(dst.shape, jnp.float32), dst,
                               num_segments=n_dst)


def _segsum_vec(P, src, dst, n_dst):
    return jax.ops.segment_sum(P[src], dst, num_segments=n_dst)


def _segsum_scalar(a, src, dst, n_dst):
    return jax.ops.segment_sum(a[src], dst, num_segments=n_dst)


def kernel(x_table, x_col, q, ei_hc, ei_rhc, ei_cs, ei_ns, ei_ds, ei_ts, params):
    c1, c2 = params['c1'], params['c2']
    u = params['lin']['W'][0]
    b_lin = params['lin']['b'][0]

    v_rhc = c2['rhc']['Wl'].T @ u
    v_ts = c2['ts']['Wl'].T @ u
    v_lin = (c2['rhc']['Wr'] + c2['ts']['Wr']).T @ u
    bias2 = (c2['rhc']['bl'] + c2['ts']['bl']) @ u
    qv = params['qp']['W'] @ q + params['qp']['b']
    qvu = qv @ u
    qr = q / jnp.maximum(jnp.linalg.norm(q), 1e-12)

    wcat_t = jnp.concatenate(
        [c1['hc']['Wl'].T, c1['ts']['Wl'].T,
         (c1['rhc']['Wr'] + c1['ts']['Wr']).T], axis=1)
    big_t, w_col = _proj_table(x_table, wcat_t, qr[:, None])
    P_t_hc, P_t_ts = big_t[:, 0:H], big_t[:, H:2 * H]
    Mt = big_t[:, 2 * H:3 * H] + (c1['rhc']['bl'] + c1['ts']['bl'])

    wcat_c = jnp.concatenate(
        [c1['cs']['Wl'].T, c1['ns']['Wl'].T, c1['ds']['Wl'].T,
         c1['rhc']['Wl'].T,
         (c1['hc']['Wr'] + c1['cs']['Wr'] + c1['ns']['Wr'] + c1['ds']['Wr']).T],
        axis=1)
    big_c = _proj_col(x_col, wcat_c)
    P_c_cs, P_c_ns = big_c[:, 0:H], big_c[:, H:2 * H]
    P_c_ds, P_c_rhc = big_c[:, 2 * H:3 * H], big_c[:, 3 * H:4 * H]
    Mc = big_c[:, 4 * H:5 * H] + (c1['hc']['bl'] + c1['cs']['bl']
                                  + c1['ns']['bl'] + c1['ds']['bl'])

    tabs = {'hc': P_t_hc, 'cs': P_c_cs, 'ns': P_c_ns, 'ds': P_c_ds,
            'rhc': P_c_rhc, 'ts': P_t_ts}

    def pad_e(ei, n, dump):
        e = ei.shape[1]
        return (jnp.pad(ei[0], (0, n - e)),
                jnp.pad(ei[1], (0, n - e), constant_values=dump))

    edges = {'hc': pad_e(ei_hc, E_HC_P, C_DUMP),
             'cs': pad_e(ei_cs, E_CC_P, C_DUMP),
             'ns': pad_e(ei_ns, E_CC_P, C_DUMP),
             'ds': pad_e(ei_ds, E_CC_P, C_DUMP),
             'rhc': pad_e(ei_rhc, E_HC_P, T_DUMP),
             'ts': (ei_ts[0], ei_ts[1])}
    S_c_p, S_t_p, invr_o, invt_o, cnt_o = _sc_segmeans(tabs, edges)
    S_c = S_c_p[:N_C]
    S_t = S_t_p[:N_T]
    inv_rhc_p = invr_o[:P_T]
    inv_ts_p = invt_o[:P_T]
    cnt_ts = cnt_o[:N_T]

    a_c = _post_col(S_c, Mc, v_rhc[:, None])[:, 0]
    vv = jnp.stack([v_ts, v_lin], axis=1)
    consts = jnp.array([[0.0, 0.0]], jnp.float32) + jnp.stack([bias2, qvu])[None, :]
    a_t2, h_base2, dis2 = _post_table(S_t, Mt, w_col, cnt_ts[:, None], vv, consts)
    a_t, h_base, dis = a_t2[:, 0], h_base2[:, 0], dis2[:, 0]

    padt = P_T - N_T
    a_t_p = jnp.pad(a_t, (0, padt))
    hb_p = jnp.pad(h_base, (0, padt))
    dis_p = jnp.pad(dis, (0, padt))
    invr_p = inv_rhc_p
    invt_p = inv_ts_p
    rr_p = jnp.pad(ei_rhc[0], (0, E_RHC_P - ei_rhc.shape[1]))
    rc_p = jnp.pad(ei_rhc[1], (0, E_RHC_P - ei_rhc.shape[1]),
                   constant_values=T_DUMP)
    v_fin = _sc_appnp(a_c, a_t_p, hb_p, dis_p, invr_p, invt_p,
                      ei_ts[0], ei_ts[1], rr_p, rc_p)
    return v_fin[:N_T] + b_lin


# speedup vs baseline: 1.8731x; 1.8731x over previous
import functools
import jax
import jax.numpy as jnp
from jax import lax
from jax.experimental import pallas as pl
from jax.experimental.pallas import tpu as pltpu
from jax.experimental.pallas import tpu_sc as plsc

H = 128
ALPHA = 0.2
K_APPNP = 10
N_T = 10000
N_C = 50000


def _proj_table_body(x_ref, wcat_ref, qr_ref, out_ref, w_ref):
    x = x_ref[...]
    out_ref[...] = jnp.dot(x, wcat_ref[...], preferred_element_type=jnp.float32)
    s = jnp.dot(x, qr_ref[...], preferred_element_type=jnp.float32)
    xn = jnp.sqrt(jnp.sum(x * x, axis=1, keepdims=True))
    w_ref[...] = jnp.maximum(s, 0.0) / jnp.maximum(xn, 1e-12)


def _proj_table(x_table, wcat, qr, blk=2000):
    nb = N_T // blk
    return pl.pallas_call(
        _proj_table_body,
        grid=(nb,),
        in_specs=[
            pl.BlockSpec((blk, 512), lambda i: (i, 0)),
            pl.BlockSpec((512, wcat.shape[1]), lambda i: (0, 0)),
            pl.BlockSpec((512, 1), lambda i: (0, 0)),
        ],
        out_specs=[
            pl.BlockSpec((blk, wcat.shape[1]), lambda i: (i, 0)),
            pl.BlockSpec((blk, 1), lambda i: (i, 0)),
        ],
        out_shape=[
            jax.ShapeDtypeStruct((N_T, wcat.shape[1]), jnp.float32),
            jax.ShapeDtypeStruct((N_T, 1), jnp.float32),
        ],
    )(x_table, wcat, qr)


def _proj_col_body(x_ref, wcat_ref, out_ref):
    out_ref[...] = jnp.dot(x_ref[...], wcat_ref[...],
                           preferred_element_type=jnp.float32)


def _proj_col(x_col, wcat, blk=2000):
    nb = N_C // blk
    return pl.pallas_call(
        _proj_col_body,
        grid=(nb,),
        in_specs=[
            pl.BlockSpec((blk, H), lambda i: (i, 0)),
            pl.BlockSpec((H, wcat.shape[1]), lambda i: (0, 0)),
        ],
        out_specs=pl.BlockSpec((blk, wcat.shape[1]), lambda i: (i, 0)),
        out_shape=jax.ShapeDtypeStruct((N_C, wcat.shape[1]), jnp.float32),
    )(x_col, wcat)


def _post_col_body(s_ref, m_ref, v_ref, a_ref):
    xc1 = jnp.maximum(s_ref[...] + m_ref[...], 0.0)
    a_ref[...] = jnp.dot(xc1, v_ref[...], preferred_element_type=jnp.float32)


def _post_col(S_c, Mc, v_rhc, blk=2000):
    nb = N_C // blk
    return pl.pallas_call(
        _post_col_body,
        grid=(nb,),
        in_specs=[
            pl.BlockSpec((blk, H), lambda i: (i, 0)),
            pl.BlockSpec((blk, H), lambda i: (i, 0)),
            pl.BlockSpec((H, 1), lambda i: (0, 0)),
        ],
        out_specs=pl.BlockSpec((blk, 1), lambda i: (i, 0)),
        out_shape=jax.ShapeDtypeStruct((N_C, 1), jnp.float32),
    )(S_c, Mc, v_rhc)


def _post_table_body(s_ref, m_ref, w_ref, cnt_ref, vv_ref, consts_ref,
                     a_ref, t_ref, d_ref):
    xt1 = jnp.maximum(s_ref[...] + m_ref[...], 0.0)
    av = jnp.dot(xt1, vv_ref[...], preferred_element_type=jnp.float32)
    a_ref[...] = av[:, 0:1]
    bias2 = consts_ref[0, 0]
    qvu = consts_ref[0, 1]
    t_ref[...] = av[:, 1:2] + bias2 + qvu * w_ref[...]
    d_ref[...] = lax.rsqrt(cnt_ref[...] + 1.0)


def _post_table(S_t, Mt, w, cnt, vv, consts, blk=2000):
    nb = N_T // blk
    return pl.pallas_call(
        _post_table_body,
        grid=(nb,),
        in_specs=[
            pl.BlockSpec((blk, H), lambda i: (i, 0)),
            pl.BlockSpec((blk, H), lambda i: (i, 0)),
            pl.BlockSpec((blk, 1), lambda i: (i, 0)),
            pl.BlockSpec((blk, 1), lambda i: (i, 0)),
            pl.BlockSpec((H, 2), lambda i: (0, 0)),
            pl.BlockSpec((1, 2), lambda i: (0, 0)),
        ],
        out_specs=[
            pl.BlockSpec((blk, 1), lambda i: (i, 0)),
            pl.BlockSpec((blk, 1), lambda i: (i, 0)),
            pl.BlockSpec((blk, 1), lambda i: (i, 0)),
        ],
        out_shape=[
            jax.ShapeDtypeStruct((N_T, 1), jnp.float32),
            jax.ShapeDtypeStruct((N_T, 1), jnp.float32),
            jax.ShapeDtypeStruct((N_T, 1), jnp.float32),
        ],
    )(S_t, Mt, w, cnt, vv, consts)


P_T = 10240
T_DUMP = 10100
E_TS = 320000
E_RHC_P = 51200
TS_CH = E_TS // 16
RHC_CH = E_RHC_P // 16
STRIPE = P_T // 16
N_SUB = 16


def _sc_appnp(a_c, a_t_p, hb_p, dis_p, invr_p, invt_p, tsr, tsc, rr_p, rc_p):
    mesh = plsc.VectorSubcoreMesh(core_axis_name="c", subcore_axis_name="s")

    @functools.partial(
        pl.kernel, mesh=mesh,
        compiler_params=pltpu.CompilerParams(needs_layout_passes=False),
        out_type=jax.ShapeDtypeStruct((P_T,), jnp.float32),
        scratch_types=[
            pltpu.VMEM((TS_CH,), jnp.int32),
            pltpu.VMEM((TS_CH,), jnp.int32),
            pltpu.VMEM((TS_CH,), jnp.float32),
            pltpu.VMEM((RHC_CH,), jnp.int32),
            pltpu.VMEM((RHC_CH,), jnp.int32),
            pltpu.VMEM((RHC_CH,), jnp.float32),
            pltpu.VMEM((P_T,), jnp.float32),
            pltpu.VMEM((P_T,), jnp.float32),
            pltpu.VMEM((P_T,), jnp.float32),
            pltpu.VMEM((P_T,), jnp.float32),
            pltpu.VMEM((STRIPE,), jnp.float32),
            pltpu.VMEM((STRIPE,), jnp.float32),
            pltpu.VMEM((STRIPE,), jnp.float32),
            pltpu.VMEM((STRIPE,), jnp.float32),
            pltpu.VMEM_SHARED((N_SUB, P_T), jnp.float32),
            pltpu.VMEM_SHARED((P_T,), jnp.float32),
            pltpu.SemaphoreType.DMA,
        ],
    )
    def k(ac_hbm, at_hbm, hb_hbm, dis_hbm, invr_hbm, invt_hbm,
          tsr_hbm, tsc_hbm, rr_hbm, rc_hbm, out_hbm,
          er, ec, en, rr, rc, gat, at, dis, val, hacc,
          hs, ds2, tmp, tmp2, slots, sv, sem):
        cid = lax.axis_index("c")
        sid = lax.axis_index("s")

        @pl.when(cid == 0)
        def _():
            so = sid * STRIPE

            def zero_hacc():
                def zb(i, _):
                    hacc[pl.ds(i * 16, 16)] = jnp.zeros((16,), jnp.float32)
                    return 0
                lax.fori_loop(0, P_T // 16, zb, 0)

            def stripe_reduce_into(dst_scale_ref, scale_from_tmp2):
                def zb(j, _):
                    tmp[pl.ds(j * 16, 16)] = jnp.zeros((16,), jnp.float32)
                    return 0
                lax.fori_loop(0, STRIPE // 16, zb, 0)
                for t in range(N_SUB):
                    pltpu.sync_copy(slots.at[t, pl.ds(so, STRIPE)], tmp2)
                    def ab(j, _):
                        s = pl.ds(j * 16, 16)
                        tmp[s] = tmp[s] + tmp2[s]
                        return 0
                    lax.fori_loop(0, STRIPE // 16, ab, 0)

            pltpu.sync_copy(tsr_hbm.at[pl.ds(sid * TS_CH, TS_CH)], er)
            pltpu.sync_copy(tsc_hbm.at[pl.ds(sid * TS_CH, TS_CH)], ec)
            pltpu.sync_copy(rr_hbm.at[pl.ds(sid * RHC_CH, RHC_CH)], rr)
            pltpu.sync_copy(rc_hbm.at[pl.ds(sid * RHC_CH, RHC_CH)], rc)
            pltpu.sync_copy(dis_hbm, dis)
            pltpu.sync_copy(at_hbm, at)

            def nb(g, _):
                s = pl.ds(g * 16, 16)
                en[s] = (plsc.load_gather(dis, [er[s]])
                         * plsc.load_gather(dis, [ec[s]]))
                return 0
            lax.fori_loop(0, TS_CH // 16, nb, 0)
            for j in range(STRIPE // 16):
                d = dis[pl.ds(so + j * 16, 16)]
                ds2[pl.ds(j * 16, 16)] = d * d

            zero_hacc()
            cps = [pltpu.async_copy(
                ac_hbm.at[rr.at[pl.ds(kk * 128, 128)]],
                gat.at[pl.ds(kk * 128, 128)], sem)
                for kk in range(RHC_CH // 128)]
            for cp in cps:
                cp.wait()

            def rb(g, _):
                s = pl.ds(g * 16, 16)
                plsc.addupdate_scatter(hacc, [rc[s]], gat[s])
                return 0
            lax.fori_loop(0, RHC_CH // 16, rb, 0)
            pltpu.sync_copy(hacc, slots.at[sid])
            plsc.subcore_barrier()
            stripe_reduce_into(None, None)
            pltpu.sync_copy(invr_hbm.at[pl.ds(so, STRIPE)], tmp2)
            for j in range(STRIPE // 16):
                s = pl.ds(j * 16, 16)
                hs[s] = tmp[s] * tmp2[s]
            plsc.subcore_barrier()

            zero_hacc()
            def tb(g, _):
                s = pl.ds(g * 16, 16)
                plsc.addupdate_scatter(hacc, [ec[s]],
                                       plsc.load_gather(at, [er[s]]))
                return 0
            lax.fori_loop(0, TS_CH // 16, tb, 0)
            pltpu.sync_copy(hacc, slots.at[sid])
            plsc.subcore_barrier()
            stripe_reduce_into(None, None)
            pltpu.sync_copy(invt_hbm.at[pl.ds(so, STRIPE)], tmp2)
            for j in range(STRIPE // 16):
                s = pl.ds(j * 16, 16)
                hs[s] = hs[s] + tmp[s] * tmp2[s]
            pltpu.sync_copy(hb_hbm.at[pl.ds(so, STRIPE)], tmp2)
            for j in range(STRIPE // 16):
                s = pl.ds(j * 16, 16)
                hs[s] = hs[s] + tmp2[s]

            pltpu.sync_copy(hs, sv.at[pl.ds(so, STRIPE)])
            plsc.subcore_barrier()
            pltpu.sync_copy(sv, val)

            for _ in range(K_APPNP):
                zero_hacc()
                def ib(g, _):
                    s = pl.ds(g * 16, 16)
                    x = plsc.load_gather(val, [er[s]]) * en[s]
                    plsc.addupdate_scatter(hacc, [ec[s]], x)
                    return 0
                lax.fori_loop(0, TS_CH // 16, ib, 0)
                pltpu.sync_copy(hacc, slots.at[sid])
                plsc.subcore_barrier()
                stripe_reduce_into(None, None)
                for j in range(STRIPE // 16):
                    s = pl.ds(j * 16, 16)
                    sj = pl.ds(so + j * 16, 16)
                    vnew = (1.0 - ALPHA) * (tmp[s] + ds2[s] * val[sj]) \
                        + ALPHA * hs[s]
                    tmp2[s] = vnew
                pltpu.sync_copy(tmp2, sv.at[pl.ds(so, STRIPE)])
                plsc.subcore_barrier()
                pltpu.sync_copy(sv, val)

            @pl.when(sid == 0)
            def _():
                pltpu.sync_copy(val, out_hbm)

    return k(a_c, a_t_p, hb_p, dis_p, invr_p, invt_p, tsr, tsc, rr_p, rc_p)


def _counts(dst, n_dst):
    return jax.ops.segment_sum(jnp.ones(dst.shape, jnp.float32), dst,
                               num_segments=n_dst)


def _segsum_vec(P, src, dst, n_dst):
    return jax.ops.segment_sum(P[src], dst, num_segments=n_dst)


def _segsum_scalar(a, src, dst, n_dst):
    return jax.ops.segment_sum(a[src], dst, num_segments=n_dst)


def kernel(x_table, x_col, q, ei_hc, ei_rhc, ei_cs, ei_ns, ei_ds, ei_ts, params):
    c1, c2 = params['c1'], params['c2']
    u = params['lin']['W'][0]
    b_lin = params['lin']['b'][0]

    v_rhc = c2['rhc']['Wl'].T @ u
    v_ts = c2['ts']['Wl'].T @ u
    v_lin = (c2['rhc']['Wr'] + c2['ts']['Wr']).T @ u
    bias2 = (c2['rhc']['bl'] + c2['ts']['bl']) @ u
    qv = params['qp']['W'] @ q + params['qp']['b']
    qvu = qv @ u
    qr = q / jnp.maximum(jnp.linalg.norm(q), 1e-12)

    wcat_t = jnp.concatenate(
        [c1['hc']['Wl'].T, c1['ts']['Wl'].T,
         (c1['rhc']['Wr'] + c1['ts']['Wr']).T], axis=1)
    big_t, w_col = _proj_table(x_table, wcat_t, qr[:, None])
    P_t_hc, P_t_ts = big_t[:, 0:H], big_t[:, H:2 * H]
    Mt = big_t[:, 2 * H:3 * H] + (c1['rhc']['bl'] + c1['ts']['bl'])

    wcat_c = jnp.concatenate(
        [c1['cs']['Wl'].T, c1['ns']['Wl'].T, c1['ds']['Wl'].T,
         c1['rhc']['Wl'].T,
         (c1['hc']['Wr'] + c1['cs']['Wr'] + c1['ns']['Wr'] + c1['ds']['Wr']).T],
        axis=1)
    big_c = _proj_col(x_col, wcat_c)
    P_c_cs, P_c_ns = big_c[:, 0:H], big_c[:, H:2 * H]
    P_c_ds, P_c_rhc = big_c[:, 2 * H:3 * H], big_c[:, 3 * H:4 * H]
    Mc = big_c[:, 4 * H:5 * H] + (c1['hc']['bl'] + c1['cs']['bl']
                                  + c1['ns']['bl'] + c1['ds']['bl'])

    cnt_hc = _counts(ei_hc[1], N_C)
    cnt_cs = _counts(ei_cs[1], N_C)
    cnt_ns = _counts(ei_ns[1], N_C)
    cnt_ds = _counts(ei_ds[1], N_C)
    cnt_rhc = _counts(ei_rhc[1], N_T)
    cnt_ts = _counts(ei_ts[1], N_T)
    inv_hc = 1.0 / jnp.maximum(cnt_hc, 1.0)
    inv_cs = 1.0 / jnp.maximum(cnt_cs, 1.0)
    inv_ns = 1.0 / jnp.maximum(cnt_ns, 1.0)
    inv_ds = 1.0 / jnp.maximum(cnt_ds, 1.0)
    inv_rhc = 1.0 / jnp.maximum(cnt_rhc, 1.0)
    inv_ts = 1.0 / jnp.maximum(cnt_ts, 1.0)

    S_c = (_segsum_vec(P_t_hc, ei_hc[0], ei_hc[1], N_C) * inv_hc[:, None]
           + _segsum_vec(P_c_cs, ei_cs[0], ei_cs[1], N_C) * inv_cs[:, None]
           + _segsum_vec(P_c_ns, ei_ns[0], ei_ns[1], N_C) * inv_ns[:, None]
           + _segsum_vec(P_c_ds, ei_ds[0], ei_ds[1], N_C) * inv_ds[:, None])
    S_t = (_segsum_vec(P_c_rhc, ei_rhc[0], ei_rhc[1], N_T) * inv_rhc[:, None]
           + _segsum_vec(P_t_ts, ei_ts[0], ei_ts[1], N_T) * inv_ts[:, None])

    a_c = _post_col(S_c, Mc, v_rhc[:, None])[:, 0]
    vv = jnp.stack([v_ts, v_lin], axis=1)
    consts = jnp.array([[0.0, 0.0]], jnp.float32) + jnp.stack([bias2, qvu])[None, :]
    a_t2, h_base2, dis2 = _post_table(S_t, Mt, w_col, cnt_ts[:, None], vv, consts)
    a_t, h_base, dis = a_t2[:, 0], h_base2[:, 0], dis2[:, 0]

    padt = P_T - N_T
    a_t_p = jnp.pad(a_t, (0, padt))
    hb_p = jnp.pad(h_base, (0, padt))
    dis_p = jnp.pad(dis, (0, padt))
    invr_p = jnp.pad(inv_rhc, (0, padt))
    invt_p = jnp.pad(inv_ts, (0, padt))
    rr_p = jnp.pad(ei_rhc[0], (0, E_RHC_P - ei_rhc.shape[1]))
    rc_p = jnp.pad(ei_rhc[1], (0, E_RHC_P - ei_rhc.shape[1]),
                   constant_values=T_DUMP)
    v_fin = _sc_appnp(a_c, a_t_p, hb_p, dis_p, invr_p, invt_p,
                      ei_ts[0], ei_ts[1], rr_p, rc_p)
    return v_fin[:N_T] + b_lin
